# baseline (speedup 1.0000x reference)
import functools

import jax
import jax.numpy as jnp
from jax import lax
from jax.experimental import pallas as pl
from jax.experimental.pallas import tpu as pltpu
from jax.experimental.pallas import tpu_sc as plsc

_PROB = 0.12


def _jitter_shift(T):
    k1, k2 = jax.random.split(jax.random.key(42))
    replace = jax.random.bernoulli(k1, _PROB, (T,))
    direction = jnp.where(jax.random.bernoulli(k2, 0.5, (T,)), 1, -1)
    idx = jnp.arange(T)
    direction = jnp.where(idx == 0, 1, direction)
    direction = jnp.where(idx == T - 1, -1, direction)
    return jnp.where(replace, direction, 0).astype(jnp.int32)


def _make_sc_jitter(rows, T):
    info = plsc.get_sparse_core_info()
    NC, NS, L = info.num_cores, info.num_subcores, info.num_lanes
    NW = NC * NS
    assert rows % NW == 0 and L == 16
    rpw = rows // NW
    r_ch = 8
    n_ch = rpw // r_ch
    chw = r_ch * T
    mesh = plsc.VectorSubcoreMesh(core_axis_name="c", subcore_axis_name="s")

    @functools.partial(
        pl.kernel,
        mesh=mesh,
        out_type=jax.ShapeDtypeStruct((rows * T,), jnp.float32),
        scratch_types=[
            pltpu.VMEM((chw + 32,), jnp.float32),
            pltpu.VMEM((chw,), jnp.float32),
            pltpu.VMEM((T,), jnp.float32),
            pltpu.VMEM((T,), jnp.float32),
        ],
    )
    def sc_jitter(x_hbm, mp_hbm, mm_hbm, out_hbm, inb, outb, mpv, mmv):
        wid = lax.axis_index("s") * NC + lax.axis_index("c")
        pltpu.sync_copy(mp_hbm, mpv)
        pltpu.sync_copy(mm_hbm, mmv)
        base = wid * (rpw * T)
        zeros16 = jnp.zeros((16,), jnp.float32)
        for ch in range(n_ch):
            start = base + ch * chw
            if ch == 0:
                @pl.when(wid == 0)
                def _():
                    inb[pl.ds(0, 16)] = zeros16
                    pltpu.sync_copy(x_hbm.at[pl.ds(0, chw + 16)],
                                    inb.at[pl.ds(16, chw + 16)])

                @pl.when(wid != 0)
                def _():
                    pltpu.sync_copy(x_hbm.at[pl.ds(start - 16, chw + 32)], inb)
            elif ch == n_ch - 1:
                @pl.when(wid == NW - 1)
                def _():
                    inb[pl.ds(chw + 16, 16)] = zeros16
                    pltpu.sync_copy(x_hbm.at[pl.ds(start - 16, chw + 16)],
                                    inb.at[pl.ds(0, chw + 16)])

                @pl.when(wid != NW - 1)
                def _():
                    pltpu.sync_copy(x_hbm.at[pl.ds(start - 16, chw + 32)], inb)
            else:
                pltpu.sync_copy(x_hbm.at[pl.ds(start - 16, chw + 32)], inb)

            @plsc.parallel_loop(0, T // 16)
            def _(j):
                t = j * 16
                mp = mpv[pl.ds(t, 16)]
                mm = mmv[pl.ds(t, 16)]
                for r in range(r_ch):
                    o = 16 + r * T + t
                    xc = inb[pl.ds(o, 16)]
                    xl = inb[pl.ds(o + 1, 16)]
                    xr = inb[pl.ds(o - 1, 16)]
                    res = jnp.where(mp > 0.5, xl,
                                    jnp.where(mm > 0.5, xr, xc))
                    outb[pl.ds(r * T + t, 16)] = res

            pltpu.sync_copy(outb, out_hbm.at[pl.ds(start, chw)])

    return sc_jitter


def kernel(quantized):
    B, C, T = quantized.shape
    rows = B * C
    d = _jitter_shift(T)
    mp = (d == 1).astype(jnp.float32)
    mm = (d == -1).astype(jnp.float32)
    x = quantized.reshape(rows * T)
    out = _make_sc_jitter(rows, T)(x, mp, mm)
    return out.reshape(B, C, T)


# speedup vs baseline: 4.9501x; 4.9501x over previous
import jax
import jax.numpy as jnp
from jax.experimental import pallas as pl
from jax.experimental.pallas import tpu as pltpu

_PROB = 0.12


def _jitter_shift(T):
    k1, k2 = jax.random.split(jax.random.key(42))
    replace = jax.random.bernoulli(k1, _PROB, (T,))
    direction = jnp.where(jax.random.bernoulli(k2, 0.5, (T,)), 1, -1)
    idx = jnp.arange(T)
    direction = jnp.where(idx == 0, 1, direction)
    direction = jnp.where(idx == T - 1, -1, direction)
    return jnp.where(replace, direction, 0).astype(jnp.int32)


def _jitter_body(d_ref, x_ref, o_ref):
    d = d_ref[...]
    m_l = d == 1
    m_r = d == -1
    rows = x_ref.shape[0]
    chunk = 8
    for j in range(rows // chunk):
        sl = pl.ds(j * chunk, chunk)
        x = x_ref[sl, :]
        xl = jnp.roll(x, -1, axis=1)
        xr = jnp.roll(x, 1, axis=1)
        o_ref[sl, :] = jnp.where(m_l, xl, jnp.where(m_r, xr, x))


def kernel(quantized):
    B, C, T = quantized.shape
    d = _jitter_shift(T).reshape(1, T)
    x = quantized.reshape(B * C, T)
    rows = B * C
    block_rows = 512
    grid = (rows // block_rows,)
    out = pl.pallas_call(
        _jitter_body,
        grid=grid,
        in_specs=[
            pl.BlockSpec((1, T), lambda i: (0, 0)),
            pl.BlockSpec((block_rows, T), lambda i: (i, 0)),
        ],
        out_specs=pl.BlockSpec((block_rows, T), lambda i: (i, 0)),
        out_shape=jax.ShapeDtypeStruct((rows, T), quantized.dtype),
        compiler_params=pltpu.CompilerParams(vmem_limit_bytes=128 * 1024 * 1024),
    )(d, x)
    return out.reshape(B, C, T)
